# Initial kernel scaffold; baseline (speedup 1.0000x reference)
import jax
import jax.numpy as jnp
from jax.experimental import pallas as pl


def kernel(x, edge_index, W1, b1, W2, b2):
    raise NotImplementedError("write your pallas kernel here")


# speedup vs baseline: 11.1491x; 11.1491x over previous
import functools

import jax
import jax.numpy as jnp
from jax import lax
from jax.experimental import pallas as pl
from jax.experimental.pallas import tpu as pltpu
from jax.experimental.pallas import tpu_sc as plsc

NC = 2
NS = 16
NW = NC * NS
L = 16

CHUNK = 128


def _pad_to(n, m):
    return ((n + m - 1) // m) * m


def _sc_mesh():
    return plsc.VectorSubcoreMesh(
        core_axis_name="c", subcore_axis_name="s", num_cores=NC, num_subcores=NS
    )


_SC_PARAMS = pltpu.CompilerParams(
    needs_layout_passes=False, use_tc_tiling_on_sc=False
)


def _make_deg_kernel(npad, ept):
    cpt = ept // CHUNK
    cb = npad // NS

    @functools.partial(
        pl.kernel,
        out_type=jax.ShapeDtypeStruct((NC, npad), jnp.float32),
        mesh=_sc_mesh(),
        compiler_params=_SC_PARAMS,
        scratch_types=[
            pltpu.VMEM((npad,), jnp.float32),
            pltpu.VMEM((CHUNK,), jnp.int32),
            pltpu.VMEM_SHARED((NS, npad), jnp.float32),
            pltpu.VMEM((NS, cb), jnp.float32),
            pltpu.VMEM((cb,), jnp.float32),
        ],
    )
    def deg_kernel(dst_hbm, out_hbm, hist, didx, staging, colblk, summed):
        c = lax.axis_index("c")
        s = lax.axis_index("s")
        wid = c * NS + s

        zero16 = jnp.zeros((L,), jnp.float32)

        def zbody(i, _):
            hist[pl.ds(i * L, L)] = zero16
            return 0

        lax.fori_loop(0, npad // L, zbody, 0)

        ones16 = jnp.ones((L,), jnp.float32)
        base = wid * ept

        def cbody(j, _):
            pltpu.sync_copy(dst_hbm.at[pl.ds(base + j * CHUNK, CHUNK)], didx)

            def gbody(g, _):
                d = didx[pl.ds(g * L, L)]
                plsc.addupdate_scatter(hist, [d], ones16)
                return 0

            lax.fori_loop(0, CHUNK // L, gbody, 0)
            return 0

        lax.fori_loop(0, cpt, cbody, 0)

        pltpu.sync_copy(hist, staging.at[s])
        plsc.subcore_barrier()

        pltpu.sync_copy(staging.at[:, pl.ds(s * cb, cb)], colblk)

        def rbody(i, _):
            v = colblk[0, pl.ds(i * L, L)]
            for t in range(1, NS):
                v = v + colblk[t, pl.ds(i * L, L)]
            summed[pl.ds(i * L, L)] = v
            return 0

        lax.fori_loop(0, cb // L, rbody, 0)
        pltpu.sync_copy(summed, out_hbm.at[c, pl.ds(s * cb, cb)])

    return deg_kernel


def _make_agg_kernel(npad, feat, ept):
    cpt = ept // CHUNK
    zrows = npad // NS // L
    wb = 128
    wchunks = npad // NS // wb

    @functools.partial(
        pl.kernel,
        out_type=jax.ShapeDtypeStruct((NC, npad, feat), jnp.float32),
        mesh=_sc_mesh(),
        compiler_params=_SC_PARAMS,
        scratch_types=[
            pltpu.VMEM((CHUNK,), jnp.int32),
            pltpu.VMEM((CHUNK,), jnp.int32),
            pltpu.VMEM((CHUNK, feat), jnp.float32),
            pltpu.VMEM_SHARED((npad, feat), jnp.float32),
            pltpu.VMEM((L, feat), jnp.float32),
            pltpu.VMEM((wb, feat), jnp.float32),
            pltpu.SemaphoreType.DMA,
        ],
    )
    def agg_kernel(g_hbm, src_hbm, dst_hbm, out_hbm,
                   sidx, didx, rows, acc, ztile, wbuf, sem):
        c = lax.axis_index("c")
        s = lax.axis_index("s")

        zero16 = jnp.zeros((L,), jnp.float32)
        for i in range(L):
            for j in range(feat // L):
                ztile[i, pl.ds(j * L, L)] = zero16

        def zbody(i, _):
            pltpu.sync_copy(ztile, acc.at[pl.ds((s * zrows + i) * L, L), :])
            return 0

        lax.fori_loop(0, zrows, zbody, 0)
        plsc.subcore_barrier()

        base = (c * NS + s) * ept

        def cbody(j, _):
            b = base + j * CHUNK
            pltpu.sync_copy(src_hbm.at[pl.ds(b, CHUNK)], sidx)
            pltpu.sync_copy(dst_hbm.at[pl.ds(b, CHUNK)], didx)
            pltpu.async_copy(g_hbm.at[sidx], rows, sem).wait()
            pltpu.sync_copy(rows, acc.at[didx], add=True)
            return 0

        lax.fori_loop(0, cpt, cbody, 0)
        plsc.subcore_barrier()

        def wbody(k, _):
            r0 = (s * wchunks + k) * wb
            pltpu.sync_copy(acc.at[pl.ds(r0, wb), :], wbuf)
            pltpu.sync_copy(wbuf, out_hbm.at[c, pl.ds(r0, wb), :])
            return 0

        lax.fori_loop(0, wchunks, wbody, 0)

    return agg_kernel


def _mm_scale_body(x_ref, w_ref, d0_ref, d1_ref, o_ref):
    deg = d0_ref[...] + d1_ref[...] + 1.0
    dis = lax.rsqrt(deg)
    h = jnp.dot(x_ref[...], w_ref[...],
                preferred_element_type=jnp.float32,
                precision=lax.Precision.HIGHEST)
    o_ref[...] = dis * h


def _mm_scale(x, w, d0, d1, blk):
    npd, din = x.shape
    feat = w.shape[1]
    return pl.pallas_call(
        _mm_scale_body,
        grid=(npd // blk,),
        in_specs=[
            pl.BlockSpec((blk, din), lambda i: (i, 0)),
            pl.BlockSpec((din, feat), lambda i: (0, 0)),
            pl.BlockSpec((blk, 1), lambda i: (i, 0)),
            pl.BlockSpec((blk, 1), lambda i: (i, 0)),
        ],
        out_specs=pl.BlockSpec((blk, feat), lambda i: (i, 0)),
        out_shape=jax.ShapeDtypeStruct((npd, feat), jnp.float32),
    )(x, w, d0, d1)


def _mid_body(p0_ref, p1_ref, g_ref, d0_ref, d1_ref, b_ref, w_ref, o_ref):
    deg = d0_ref[...] + d1_ref[...] + 1.0
    dis = lax.rsqrt(deg)
    z = dis * (p0_ref[...] + p1_ref[...] + g_ref[...]) + b_ref[...]
    z = jnp.maximum(z, 0.0)
    h = jnp.dot(z, w_ref[...],
                preferred_element_type=jnp.float32,
                precision=lax.Precision.HIGHEST)
    o_ref[...] = dis * h


def _mid(p0, p1, g, d0, d1, b, w, blk):
    npd, din = g.shape
    feat = w.shape[1]
    return pl.pallas_call(
        _mid_body,
        grid=(npd // blk,),
        in_specs=[
            pl.BlockSpec((blk, din), lambda i: (i, 0)),
            pl.BlockSpec((blk, din), lambda i: (i, 0)),
            pl.BlockSpec((blk, din), lambda i: (i, 0)),
            pl.BlockSpec((blk, 1), lambda i: (i, 0)),
            pl.BlockSpec((blk, 1), lambda i: (i, 0)),
            pl.BlockSpec((din,), lambda i: (0,)),
            pl.BlockSpec((din, feat), lambda i: (0, 0)),
        ],
        out_specs=pl.BlockSpec((blk, feat), lambda i: (i, 0)),
        out_shape=jax.ShapeDtypeStruct((npd, feat), jnp.float32),
    )(p0, p1, g, d0, d1, b, w)


def _final_body(q0_ref, q1_ref, g_ref, d0_ref, d1_ref, b_ref, o_ref):
    deg = d0_ref[...] + d1_ref[...] + 1.0
    dis = lax.rsqrt(deg)
    o_ref[...] = dis * (q0_ref[...] + q1_ref[...] + g_ref[...]) + b_ref[...]


def _final(q0, q1, g, d0, d1, b, blk):
    npd, feat = g.shape
    return pl.pallas_call(
        _final_body,
        grid=(npd // blk,),
        in_specs=[
            pl.BlockSpec((blk, feat), lambda i: (i, 0)),
            pl.BlockSpec((blk, feat), lambda i: (i, 0)),
            pl.BlockSpec((blk, feat), lambda i: (i, 0)),
            pl.BlockSpec((blk, 1), lambda i: (i, 0)),
            pl.BlockSpec((blk, 1), lambda i: (i, 0)),
            pl.BlockSpec((feat,), lambda i: (0,)),
        ],
        out_specs=pl.BlockSpec((blk, feat), lambda i: (i, 0)),
        out_shape=jax.ShapeDtypeStruct((npd, feat), jnp.float32),
    )(q0, q1, g, d0, d1, b)


def kernel(x, edge_index, W1, b1, W2, b2):
    n, d = x.shape
    e = edge_index.shape[1]
    npad = _pad_to(n, 1024)
    ept = _pad_to(e, NW * CHUNK) // NW
    epad = ept * NW

    src = jnp.concatenate(
        [edge_index[0], jnp.zeros((epad - e,), jnp.int32)])
    dst = jnp.concatenate(
        [edge_index[1], jnp.full((epad - e,), n, jnp.int32)])
    xp = jnp.concatenate(
        [x, jnp.zeros((npad - n, d), jnp.float32)])

    deg_parts = _make_deg_kernel(npad, ept)(dst)
    d0 = deg_parts[0].reshape(npad, 1)
    d1 = deg_parts[1].reshape(npad, 1)

    blk = 1024
    g1 = _mm_scale(xp, W1, d0, d1, blk)

    agg1 = _make_agg_kernel(npad, W1.shape[1], ept)(g1, src, dst)
    g2 = _mid(agg1[0], agg1[1], g1, d0, d1, b1, W2, blk)

    agg2 = _make_agg_kernel(npad, W2.shape[1], ept)(g2, src, dst)
    out = _final(agg2[0], agg2[1], g2, d0, d1, b2, blk)
    return out[:n]
